# Initial kernel scaffold; baseline (speedup 1.0000x reference)
import jax
import jax.numpy as jnp
from jax.experimental import pallas as pl


def kernel(nfeats, edge_index, efeats, W_apply, b_apply, W_edge, b_edge):
    raise NotImplementedError("write your pallas kernel here")


# speedup vs baseline: 5.0294x; 5.0294x over previous
import functools

import jax
import jax.numpy as jnp
from jax import lax
from jax.experimental import pallas as pl
from jax.experimental.pallas import tpu as pltpu
from jax.experimental.pallas import tpu_sc as plsc

NC = 2
NS = 16
NW = NC * NS
LANES = 16
B = 128

F32 = jnp.float32


def _sc_aggregate(nfA, nfB, efeats_pad, u3, v3, n_pad):
    nblk = u3.shape[1]
    hdim = nfA.shape[1]
    edim = efeats_pad.shape[1]
    rows_per_tile = n_pad // NS
    nzb = rows_per_tile // B

    mesh = plsc.VectorSubcoreMesh(core_axis_name="c", subcore_axis_name="s")
    out_types = (
        jax.ShapeDtypeStruct((NC, n_pad, hdim), F32),
        jax.ShapeDtypeStruct((n_pad, edim), F32),
        jax.ShapeDtypeStruct((n_pad, edim), F32),
    )

    @functools.partial(
        pl.kernel,
        out_type=out_types,
        mesh=mesh,
        scratch_types=[
            pltpu.VMEM((nblk, B), jnp.int32),
            pltpu.VMEM((nblk, B), jnp.int32),
            pltpu.VMEM((B, hdim), F32),
            pltpu.VMEM((B, edim), F32),
            pltpu.VMEM((B, edim), F32),
            pltpu.VMEM((B, hdim), F32),
            pltpu.VMEM((B, edim), F32),
            pltpu.VMEM_SHARED((n_pad, hdim), F32),
            pltpu.VMEM_SHARED((n_pad, edim), F32),
            pltpu.SemaphoreType.DMA,
            pltpu.SemaphoreType.DMA,
        ],
        compiler_params=pltpu.CompilerParams(use_tc_tiling_on_sc=False),
    )
    def agg(nfa_hbm, nfb_hbm, ef_hbm, u_hbm, v_hbm, outN, outE, outD,
            u_v, v_v, rowbuf, ebuf, ones_v, zbufN, zbufE,
            accN, acc2, sem1, sem2):
        cid = lax.axis_index("c")
        sid = lax.axis_index("s")

        @pl.loop(0, B)
        def _(r):
            @pl.loop(0, hdim // LANES)
            def _(cc):
                zbufN.at[pl.ds(r, 1), pl.ds(cc * LANES, LANES)][...] = (
                    jnp.zeros((1, LANES), F32))
            zbufE.at[pl.ds(r, 1), pl.ds(0, LANES)][...] = (
                jnp.zeros((1, LANES), F32))
            ones_v.at[pl.ds(r, 1), pl.ds(0, LANES)][...] = (
                jnp.ones((1, LANES), F32))

        @pl.loop(0, nzb)
        def _(kk):
            base = sid * rows_per_tile + kk * B
            pltpu.sync_copy(zbufN, accN.at[pl.ds(base, B)])
            pltpu.sync_copy(zbufE, acc2.at[pl.ds(base, B)])
        plsc.subcore_barrier()

        pltpu.sync_copy(u_hbm.at[sid], u_v)
        pltpu.sync_copy(v_hbm.at[sid], v_v)
        ebase = sid * nblk * B

        @pl.when(cid == 0)
        def _():
            @pl.loop(0, nblk)
            def _(j):
                g1 = pltpu.async_copy(nfa_hbm.at[u_v.at[j]], rowbuf, sem1)
                g2 = pltpu.async_copy(ef_hbm.at[pl.ds(ebase + j * B, B)],
                                      ebuf, sem2)
                g1.wait()
                g2.wait()
                pltpu.sync_copy(rowbuf, accN.at[v_v.at[j]], add=True)
                pltpu.sync_copy(ebuf, acc2.at[v_v.at[j]], add=True)

        @pl.when(cid == 1)
        def _():
            @pl.loop(0, nblk)
            def _(j):
                g1 = pltpu.async_copy(nfb_hbm.at[u_v.at[j]], rowbuf, sem1)
                g1.wait()
                pltpu.sync_copy(rowbuf, accN.at[v_v.at[j]], add=True)
                pltpu.sync_copy(ones_v, acc2.at[v_v.at[j]], add=True)

        plsc.subcore_barrier()
        rb = sid * rows_per_tile
        pltpu.sync_copy(accN.at[pl.ds(rb, rows_per_tile)],
                        outN.at[cid, pl.ds(rb, rows_per_tile)])

        @pl.when(cid == 0)
        def _():
            pltpu.sync_copy(acc2.at[pl.ds(rb, rows_per_tile)],
                            outE.at[pl.ds(rb, rows_per_tile)])

        @pl.when(cid == 1)
        def _():
            pltpu.sync_copy(acc2.at[pl.ds(rb, rows_per_tile)],
                            outD.at[pl.ds(rb, rows_per_tile)])

    return agg(nfA, nfB, efeats_pad, u3, v3)


def _tc_node_update(accN, accE, accD, nfeats_pad, W1, W2, W3, We1, We2, bap):
    n_pad, ndim = nfeats_pad.shape
    hdim = accN.shape[2]
    edim = accE.shape[1]
    eout = We1.shape[0]
    R = 1024
    grid = (n_pad // R,)

    def body(aN, aE, aD, nf, w1, w2, w3, we1, we2, b, h_ref, a_ref, b_ref):
        dn = (((1,), (1,)), ((), ()))
        deg = jnp.maximum(aD[:, 0:1], 1.0)
        w2v = w2[...]
        neigh = (
            lax.dot_general(aN[0], w2v[:, :hdim], dn,
                            preferred_element_type=F32)
            + lax.dot_general(aN[1], w2v[:, hdim:], dn,
                              preferred_element_type=F32)
            + lax.dot_general(aE[...], w3[...], dn,
                              preferred_element_type=F32)
        ) / deg
        h = lax.dot_general(nf[...], w1[...], dn, preferred_element_type=F32)
        h = jnp.maximum(h + neigh + b[...], 0.0)
        h_ref[...] = h
        a_ref[...] = lax.dot_general(h, we1[...], dn,
                                     preferred_element_type=F32)
        b_ref[...] = lax.dot_general(h, we2[...], dn,
                                     preferred_element_type=F32)

    full = lambda shape: pl.BlockSpec(shape, lambda i: tuple(0 for _ in shape))
    return pl.pallas_call(
        body,
        grid=grid,
        in_specs=[
            pl.BlockSpec((NC, R, hdim), lambda i: (0, i, 0)),
            pl.BlockSpec((R, edim), lambda i: (i, 0)),
            pl.BlockSpec((R, edim), lambda i: (i, 0)),
            pl.BlockSpec((R, ndim), lambda i: (i, 0)),
            full(W1.shape), full(W2.shape), full(W3.shape),
            full(We1.shape), full(We2.shape), full(bap.shape),
        ],
        out_specs=[
            pl.BlockSpec((R, ndim), lambda i: (i, 0)),
            pl.BlockSpec((R, eout), lambda i: (i, 0)),
            pl.BlockSpec((R, eout), lambda i: (i, 0)),
        ],
        out_shape=[
            jax.ShapeDtypeStruct((n_pad, ndim), F32),
            jax.ShapeDtypeStruct((n_pad, eout), F32),
            jax.ShapeDtypeStruct((n_pad, eout), F32),
        ],
    )(accN, accE, accD, nfeats_pad, W1, W2, W3, We1, We2, bap)


def _sc_edge_gather(A, Bm, u3, v3):
    nblk = u3.shape[1]
    eout = A.shape[1]
    e_pad = NW * nblk * B
    mesh = plsc.VectorSubcoreMesh(core_axis_name="c", subcore_axis_name="s")
    out_types = (
        jax.ShapeDtypeStruct((e_pad, eout), F32),
        jax.ShapeDtypeStruct((e_pad, eout), F32),
    )

    @functools.partial(
        pl.kernel,
        out_type=out_types,
        mesh=mesh,
        scratch_types=[
            pltpu.VMEM((nblk, B), jnp.int32),
            pltpu.VMEM((nblk, B), jnp.int32),
            pltpu.VMEM((B, eout), F32),
            pltpu.VMEM((B, eout), F32),
            pltpu.SemaphoreType.DMA,
            pltpu.SemaphoreType.DMA,
        ],
        compiler_params=pltpu.CompilerParams(use_tc_tiling_on_sc=False),
    )
    def eg(a_hbm, b_hbm, u_hbm, v_hbm, ga_hbm, gb_hbm,
           u_v, v_v, abuf, bbuf, sem1, sem2):
        cid = lax.axis_index("c")
        sid = lax.axis_index("s")
        wid = cid * NS + sid
        pltpu.sync_copy(u_hbm.at[wid], u_v)
        pltpu.sync_copy(v_hbm.at[wid], v_v)
        ebase = wid * nblk * B

        @pl.loop(0, nblk)
        def _(j):
            g1 = pltpu.async_copy(a_hbm.at[u_v.at[j]], abuf, sem1)
            g2 = pltpu.async_copy(b_hbm.at[v_v.at[j]], bbuf, sem2)
            g1.wait()
            g2.wait()
            pltpu.sync_copy(abuf, ga_hbm.at[pl.ds(ebase + j * B, B)])
            pltpu.sync_copy(bbuf, gb_hbm.at[pl.ds(ebase + j * B, B)])

    return eg(A, Bm, u3, v3)


def _tc_edge_update(Ga, Gb, bias_row):
    rows = Ga.shape[0]
    blk = rows // 16

    def body(ga, gb, b, o_ref):
        o_ref[...] = jnp.maximum(ga[...] + gb[...] + b[...], 0.0)

    return pl.pallas_call(
        body,
        grid=(16,),
        in_specs=[
            pl.BlockSpec((blk, 128), lambda i: (i, 0)),
            pl.BlockSpec((blk, 128), lambda i: (i, 0)),
            pl.BlockSpec((1, 128), lambda i: (0, 0)),
        ],
        out_specs=pl.BlockSpec((blk, 128), lambda i: (i, 0)),
        out_shape=jax.ShapeDtypeStruct((rows, 128), F32),
    )(Ga, Gb, bias_row)


def kernel(nfeats, edge_index, efeats, W_apply, b_apply, W_edge, b_edge):
    N, ndim = nfeats.shape
    E, edim = efeats.shape
    nout = W_apply.shape[0]
    eout = W_edge.shape[0]
    hdim = ndim // 2

    u = edge_index[0].astype(jnp.int32)
    v = edge_index[1].astype(jnp.int32)

    nblk1 = -(-E // (NS * B))
    e_pad1 = NS * nblk1 * B
    n_pad = -(-(N + 1) // (NS * B)) * (NS * B)
    u3 = jnp.pad(u, (0, e_pad1 - E)).reshape(NS, nblk1, B)
    v3 = jnp.pad(v, (0, e_pad1 - E), constant_values=N).reshape(NS, nblk1, B)
    efeats_pad = jnp.pad(efeats, ((0, e_pad1 - E), (0, 0)))
    nfA = nfeats[:, :hdim]
    nfB = nfeats[:, hdim:]

    accN, accE, accD = _sc_aggregate(nfA, nfB, efeats_pad, u3, v3, n_pad)

    nfeats_pad = jnp.pad(nfeats, ((0, n_pad - N), (0, 0)))
    W1 = W_apply[:, :ndim]
    W2 = W_apply[:, ndim:2 * ndim]
    W3 = W_apply[:, 2 * ndim:]
    We1 = W_edge[:, :nout]
    We2 = W_edge[:, nout:]
    h_pad, A, Bm = _tc_node_update(accN, accE, accD, nfeats_pad,
                                   W1, W2, W3, We1, We2,
                                   b_apply.reshape(1, nout))

    nblk2 = -(-E // (NW * B))
    e_pad2 = NW * nblk2 * B
    u32 = jnp.pad(u, (0, e_pad2 - E)).reshape(NW, nblk2, B)
    v32 = jnp.pad(v, (0, e_pad2 - E)).reshape(NW, nblk2, B)

    Ga, Gb = _sc_edge_gather(A, Bm, u32, v32)
    rows = e_pad2 * eout // 128
    bias_row = jnp.tile(b_edge, 128 // eout).reshape(1, 128)
    he = _tc_edge_update(Ga.reshape(rows, 128), Gb.reshape(rows, 128),
                         bias_row)

    h_nodes_new = h_pad[:N]
    h_edges_new = he.reshape(e_pad2, eout)[:E]
    return (h_nodes_new, h_edges_new)


# speedup vs baseline: 6.1473x; 1.2223x over previous
import functools

import jax
import jax.numpy as jnp
from jax import lax
from jax.experimental import pallas as pl
from jax.experimental.pallas import tpu as pltpu
from jax.experimental.pallas import tpu_sc as plsc

NC = 2
NS = 16
LANES = 16
B = 128

F32 = jnp.float32


def _sc_aggregate(nf2, efeats, u2d, v2d, n_pad, n_edges, nblk):
    hdim = nf2.shape[1]
    edim = efeats.shape[1]
    rows_per_tile = n_pad // NS
    nzb = rows_per_tile // B

    mesh = plsc.VectorSubcoreMesh(core_axis_name="c", subcore_axis_name="s")
    out_types = (
        jax.ShapeDtypeStruct((NC, n_pad, hdim), F32),
        jax.ShapeDtypeStruct((NC, n_pad, edim), F32),
    )

    @functools.partial(
        pl.kernel,
        out_type=out_types,
        mesh=mesh,
        scratch_types=[
            pltpu.VMEM((nblk, B), jnp.int32),
            pltpu.VMEM((nblk, B), jnp.int32),
            pltpu.VMEM((B, hdim), F32),
            pltpu.VMEM((B, hdim), F32),
            pltpu.VMEM((B, edim), F32),
            pltpu.VMEM((B, edim), F32),
            pltpu.VMEM((B, edim), F32),
            pltpu.VMEM((B, hdim), F32),
            pltpu.VMEM((B, edim), F32),
            pltpu.VMEM_SHARED((n_pad, hdim), F32),
            pltpu.VMEM_SHARED((n_pad, edim), F32),
            pltpu.SemaphoreType.DMA,
            pltpu.SemaphoreType.DMA,
            pltpu.SemaphoreType.DMA,
            pltpu.SemaphoreType.DMA,
        ],
        compiler_params=pltpu.CompilerParams(use_tc_tiling_on_sc=False),
    )
    def agg(nf_hbm, ef_hbm, u_hbm, v_hbm, outN, outE,
            u_v, v_v, row0, row1, ebuf0, ebuf1, ones_v, zbufN, zbufE,
            accN, acc2, gsem0, gsem1, ssem0, ssem1):
        cid = lax.axis_index("c")
        sid = lax.axis_index("s")
        rows = (row0, row1)
        ebufs = (ebuf0, ebuf1)
        gsems = (gsem0, gsem1)
        ssems = (ssem0, ssem1)
        on_c0 = cid == 0

        @pl.loop(0, B)
        def _(r):
            @pl.loop(0, hdim // LANES)
            def _(cc):
                zbufN.at[pl.ds(r, 1), pl.ds(cc * LANES, LANES)][...] = (
                    jnp.zeros((1, LANES), F32))
            zbufE.at[pl.ds(r, 1), pl.ds(0, LANES)][...] = (
                jnp.zeros((1, LANES), F32))
            ones_v.at[pl.ds(r, 1), pl.ds(0, LANES)][...] = (
                jnp.ones((1, LANES), F32))

        @pl.loop(0, nzb)
        def _(kk):
            base = sid * rows_per_tile + kk * B
            pltpu.sync_copy(zbufN, accN.at[pl.ds(base, B)])
            pltpu.sync_copy(zbufE, acc2.at[pl.ds(base, B)])

        pltpu.sync_copy(u_hbm.at[pl.ds(sid * nblk, nblk)], u_v)
        pltpu.sync_copy(v_hbm.at[pl.ds(sid * nblk, nblk)], v_v)
        cvec = jnp.full((1, LANES), 0, jnp.int32) + cid

        @pl.loop(0, nblk)
        def _(r):
            @pl.loop(0, B // LANES)
            def _(cc):
                slc = (pl.ds(r, 1), pl.ds(cc * LANES, LANES))
                u_v.at[*slc][...] = u_v.at[*slc][...] * 2 + cvec

        plsc.subcore_barrier()
        ebase = sid * nblk * B

        def my_ef(j):
            return on_c0 & (ebase + j * B + B <= n_edges)

        def g_start(j, b):
            pltpu.async_copy(nf_hbm.at[u_v.at[j]], rows[b], gsems[b])

            @pl.when(my_ef(j))
            def _():
                pltpu.async_copy(ef_hbm.at[pl.ds(ebase + j * B, B)],
                                 ebufs[b], gsems[b])

        def g_wait(j, b):
            pltpu.make_async_copy(nf_hbm.at[u_v.at[j]], rows[b],
                                  gsems[b]).wait()

            @pl.when(my_ef(j))
            def _():
                pltpu.make_async_copy(ef_hbm.at[pl.ds(ebase + j * B, B)],
                                      ebufs[b], gsems[b]).wait()

        def s_start(j, b):
            pltpu.async_copy(rows[b], accN.at[v_v.at[j]], ssems[b], add=True)

            @pl.when(my_ef(j))
            def _():
                pltpu.async_copy(ebufs[b], acc2.at[v_v.at[j]], ssems[b],
                                 add=True)

            @pl.when(~on_c0)
            def _():
                pltpu.async_copy(ones_v, acc2.at[v_v.at[j]], ssems[b],
                                 add=True)

        def s_wait(j, b):
            pltpu.make_async_copy(rows[b], accN.at[v_v.at[j]],
                                  ssems[b]).wait()

            @pl.when(my_ef(j))
            def _():
                pltpu.make_async_copy(ebufs[b], acc2.at[v_v.at[j]],
                                      ssems[b]).wait()

            @pl.when(~on_c0)
            def _():
                pltpu.make_async_copy(ones_v, acc2.at[v_v.at[j]],
                                      ssems[b]).wait()

        g_start(0, 0)

        @pl.loop(0, nblk // 2)
        def _(jo):
            j0 = 2 * jo
            j1 = j0 + 1
            g_start(j1, 1)
            g_wait(j0, 0)
            s_start(j0, 0)
            s_wait(j0, 0)

            @pl.when(jo < nblk // 2 - 1)
            def _():
                g_start(j0 + 2, 0)

            g_wait(j1, 1)
            s_start(j1, 1)
            s_wait(j1, 1)

        plsc.subcore_barrier()
        rb = sid * rows_per_tile
        pltpu.sync_copy(accN.at[pl.ds(rb, rows_per_tile)],
                        outN.at[cid, pl.ds(rb, rows_per_tile)])
        pltpu.sync_copy(acc2.at[pl.ds(rb, rows_per_tile)],
                        outE.at[cid, pl.ds(rb, rows_per_tile)])

    return agg(nf2, efeats, u2d, v2d)


def _tc_node_update(accN, accE, nfeats, W1, W2, W3, We1, We2, bap):
    n, ndim = nfeats.shape
    hdim = accN.shape[2]
    edim = accE.shape[2]
    eout = We1.shape[0]
    R = 1000
    grid = (n // R,)

    def body(aN, aE, nf, w1, w2, w3, we1, we2, b, h_ref, a_ref, b_ref):
        dn = (((1,), (1,)), ((), ()))
        deg = jnp.maximum(aE[1, :, 0:1], 1.0)
        w2v = w2[...]
        neigh = (
            lax.dot_general(aN[0], w2v[:, :hdim], dn,
                            preferred_element_type=F32)
            + lax.dot_general(aN[1], w2v[:, hdim:], dn,
                              preferred_element_type=F32)
            + lax.dot_general(aE[0], w3[...], dn, preferred_element_type=F32)
        ) / deg
        h = lax.dot_general(nf[...], w1[...], dn, preferred_element_type=F32)
        h = jnp.maximum(h + neigh + b[...], 0.0)
        h_ref[...] = h
        a_ref[...] = lax.dot_general(h, we1[...], dn,
                                     preferred_element_type=F32)
        b_ref[...] = lax.dot_general(h, we2[...], dn,
                                     preferred_element_type=F32)

    full = lambda shape: pl.BlockSpec(shape, lambda i: tuple(0 for _ in shape))
    return pl.pallas_call(
        body,
        grid=grid,
        in_specs=[
            pl.BlockSpec((NC, R, hdim), lambda i: (0, i, 0)),
            pl.BlockSpec((NC, R, edim), lambda i: (0, i, 0)),
            pl.BlockSpec((R, ndim), lambda i: (i, 0)),
            full(W1.shape), full(W2.shape), full(W3.shape),
            full(We1.shape), full(We2.shape), full(bap.shape),
        ],
        out_specs=[
            pl.BlockSpec((R, ndim), lambda i: (i, 0)),
            pl.BlockSpec((R, eout), lambda i: (i, 0)),
            pl.BlockSpec((R, eout), lambda i: (i, 0)),
        ],
        out_shape=[
            jax.ShapeDtypeStruct((n, ndim), F32),
            jax.ShapeDtypeStruct((n, eout), F32),
            jax.ShapeDtypeStruct((n, eout), F32),
        ],
    )(accN, accE, nfeats, W1, W2, W3, We1, We2, bap)


def _sc_edge_gather(A, Bm, u2d, v2d, nblk):
    eout = A.shape[1]
    e_pad = NS * nblk * B
    mesh = plsc.VectorSubcoreMesh(core_axis_name="c", subcore_axis_name="s")
    out_types = (
        jax.ShapeDtypeStruct((e_pad, eout), F32),
        jax.ShapeDtypeStruct((e_pad, eout), F32),
    )

    @functools.partial(
        pl.kernel,
        out_type=out_types,
        mesh=mesh,
        scratch_types=[
            pltpu.VMEM((nblk, B), jnp.int32),
            pltpu.VMEM((B, eout), F32),
            pltpu.VMEM((B, eout), F32),
            pltpu.SemaphoreType.DMA,
            pltpu.SemaphoreType.DMA,
            pltpu.SemaphoreType.DMA,
            pltpu.SemaphoreType.DMA,
        ],
        compiler_params=pltpu.CompilerParams(use_tc_tiling_on_sc=False),
    )
    def eg(a_hbm, b_hbm, u_hbm, v_hbm, ga_hbm, gb_hbm,
           i_v, buf0, buf1, gsem0, gsem1, wsem0, wsem1):
        cid = lax.axis_index("c")
        sid = lax.axis_index("s")
        on_c0 = cid == 0
        bufs = (buf0, buf1)
        gsems = (gsem0, gsem1)
        wsems = (wsem0, wsem1)
        ebase = sid * nblk * B

        @pl.when(on_c0)
        def _():
            pltpu.sync_copy(u_hbm.at[pl.ds(sid * nblk, nblk)], i_v)

        @pl.when(~on_c0)
        def _():
            pltpu.sync_copy(v_hbm.at[pl.ds(sid * nblk, nblk)], i_v)

        def g_start(j, b):
            @pl.when(on_c0)
            def _():
                pltpu.async_copy(a_hbm.at[i_v.at[j]], bufs[b], gsems[b])

            @pl.when(~on_c0)
            def _():
                pltpu.async_copy(b_hbm.at[i_v.at[j]], bufs[b], gsems[b])

        def g_wait(j, b):
            pltpu.make_async_copy(a_hbm.at[i_v.at[j]], bufs[b],
                                  gsems[b]).wait()

        def w_start(j, b):
            @pl.when(on_c0)
            def _():
                pltpu.async_copy(bufs[b], ga_hbm.at[pl.ds(ebase + j * B, B)],
                                 wsems[b])

            @pl.when(~on_c0)
            def _():
                pltpu.async_copy(bufs[b], gb_hbm.at[pl.ds(ebase + j * B, B)],
                                 wsems[b])

        def w_wait(j, b):
            pltpu.make_async_copy(bufs[b],
                                  ga_hbm.at[pl.ds(ebase + j * B, B)],
                                  wsems[b]).wait()

        g_start(0, 0)

        @pl.loop(0, nblk // 2)
        def _(jo):
            j0 = 2 * jo
            j1 = j0 + 1
            g_start(j1, 1)
            g_wait(j0, 0)
            w_start(j0, 0)
            w_wait(j0, 0)

            @pl.when(jo < nblk // 2 - 1)
            def _():
                g_start(j0 + 2, 0)

            g_wait(j1, 1)
            w_start(j1, 1)
            w_wait(j1, 1)

    return eg(A, Bm, u2d, v2d)


def _tc_edge_update(Ga, Gb, bias_row, n_rows):
    blk = n_rows // 10

    def body(ga, gb, b, o_ref):
        o_ref[...] = jnp.maximum(ga[...] + gb[...] + b[...], 0.0)

    return pl.pallas_call(
        body,
        grid=(10,),
        in_specs=[
            pl.BlockSpec((blk, 128), lambda i: (i, 0)),
            pl.BlockSpec((blk, 128), lambda i: (i, 0)),
            pl.BlockSpec((1, 128), lambda i: (0, 0)),
        ],
        out_specs=pl.BlockSpec((blk, 128), lambda i: (i, 0)),
        out_shape=jax.ShapeDtypeStruct((n_rows, 128), F32),
    )(Ga, Gb, bias_row)


def kernel(nfeats, edge_index, efeats, W_apply, b_apply, W_edge, b_edge):
    N, ndim = nfeats.shape
    E, edim = efeats.shape
    nout = W_apply.shape[0]
    eout = W_edge.shape[0]
    hdim = ndim // 2

    u = edge_index[0].astype(jnp.int32)
    v = edge_index[1].astype(jnp.int32)

    nblk = -(-E // (NS * B))
    nblk += nblk % 2
    e_pad = NS * nblk * B
    n_pad = -(-(N + 1) // (NS * B)) * (NS * B)
    u2d = jnp.pad(u, (0, e_pad - E)).reshape(NS * nblk, B)
    v2d = jnp.pad(v, (0, e_pad - E), constant_values=N).reshape(NS * nblk, B)
    nf2 = nfeats.reshape(2 * N, hdim)

    accN, accE = _sc_aggregate(nf2, efeats, u2d, v2d, n_pad, E, nblk)

    W1 = W_apply[:, :ndim]
    W2 = W_apply[:, ndim:2 * ndim]
    W3 = W_apply[:, 2 * ndim:]
    We1 = W_edge[:, :nout]
    We2 = W_edge[:, nout:]
    h, A, Bm = _tc_node_update(accN, accE, nfeats,
                               W1, W2, W3, We1, We2,
                               b_apply.reshape(1, nout))

    Ga, Gb = _sc_edge_gather(A, Bm, u2d, v2d, nblk)
    rows = e_pad * eout // 128
    n_rows = E * eout // 128
    bias_row = jnp.tile(b_edge, 128 // eout).reshape(1, 128)
    he = _tc_edge_update(Ga.reshape(rows, 128), Gb.reshape(rows, 128),
                         bias_row, n_rows)

    h_edges_new = he.reshape(E, eout)
    return (h, h_edges_new)
